# Initial kernel scaffold; baseline (speedup 1.0000x reference)
import jax
import jax.numpy as jnp
from jax import lax
from jax.experimental import pallas as pl
from jax.experimental.pallas import tpu as pltpu
from jax.experimental.pallas import tpu_sc as plsc

B = 16384
DIM = 32
NEG = 20
LANES = 16

NC = 2
NS = 16
NW = NC * NS

B_PER_W = B // NW
CB = 128
NCH = B_PER_W // CB
NEG_CB = CB * NEG
NEG_STREAMS = NEG_CB // 128


def _w2v_body(in_hbm, ctx_hbm, neg_hbm, enc_hbm, dec_hbm,
              pos_out, neg_out,
              widx, cidx, nidx, wrows, crows, nrows, posbuf, negbuf, sem):
  wid = lax.axis_index("s") * NC + lax.axis_index("c")

  def chunk_body(c, _):
    base = pl.multiple_of((wid * NCH + c) * CB, CB)
    pltpu.sync_copy(in_hbm.at[pl.ds(base, CB)], widx)
    pltpu.sync_copy(ctx_hbm.at[pl.ds(base, CB)], cidx)
    nrow = pl.multiple_of((wid * NCH + c) * NEG_STREAMS, NEG_STREAMS)
    pltpu.sync_copy(neg_hbm.at[pl.ds(nrow, NEG_STREAMS)], nidx)

    handles = [
        pltpu.async_copy(enc_hbm.at[widx], wrows, sem),
        pltpu.async_copy(dec_hbm.at[cidx], crows, sem),
    ]
    for j in range(NEG_STREAMS):
      handles.append(
          pltpu.async_copy(dec_hbm.at[nidx.at[j]],
                           nrows.at[pl.ds(j * 128, 128)], sem))
    for h in handles:
      h.wait()

    dsplat = [jnp.full((LANES,), d, jnp.int32) for d in range(DIM)]

    def group_body(g, _):
      lb = pl.multiple_of(g * LANES, LANES)
      row_ids = lb + lax.iota(jnp.int32, LANES)
      w = [plsc.load_gather(wrows, [row_ids, dsplat[d]]) for d in range(DIM)]
      acc = w[0] * plsc.load_gather(crows, [row_ids, dsplat[0]])
      for d in range(1, DIM):
        acc = acc + w[d] * plsc.load_gather(crows, [row_ids, dsplat[d]])
      posbuf[pl.ds(lb, LANES)] = acc
      nrow_base = row_ids * NEG
      for k in range(NEG):
        r = nrow_base + k
        acc = w[0] * plsc.load_gather(nrows, [r, dsplat[0]])
        for d in range(1, DIM):
          acc = acc + w[d] * plsc.load_gather(nrows, [r, dsplat[d]])
        plsc.store_scatter(negbuf, [r], acc)
      return 0

    lax.fori_loop(0, CB // LANES, group_body, 0)

    pltpu.sync_copy(posbuf, pos_out.at[pl.ds(base, CB)])
    pltpu.sync_copy(negbuf, neg_out.at[pl.ds(base * NEG, NEG_CB)])
    return 0

  lax.fori_loop(0, NCH, chunk_body, 0)


@jax.jit
def _w2v_call(in_flat, ctx_flat, neg2d, enc, dec):
  mesh = plsc.VectorSubcoreMesh(core_axis_name="c", subcore_axis_name="s")
  kern = pl.kernel(
      _w2v_body,
      out_type=(
          jax.ShapeDtypeStruct((B,), jnp.float32),
          jax.ShapeDtypeStruct((B * NEG,), jnp.float32),
      ),
      mesh=mesh,
      scratch_types=[
          pltpu.VMEM((CB,), jnp.int32),
          pltpu.VMEM((CB,), jnp.int32),
          pltpu.VMEM((NEG_STREAMS, 128), jnp.int32),
          pltpu.VMEM((CB, DIM), jnp.float32),
          pltpu.VMEM((CB, DIM), jnp.float32),
          pltpu.VMEM((NEG_CB, DIM), jnp.float32),
          pltpu.VMEM((CB,), jnp.float32),
          pltpu.VMEM((NEG_CB,), jnp.float32),
          pltpu.SemaphoreType.DMA,
      ],
  )
  return kern(in_flat, ctx_flat, neg2d, enc, dec)


def kernel(input_tokens, ctx_tokens, neg_tokens, encoder_weight, decoder_weight):
  in_flat = input_tokens.reshape(B).astype(jnp.int32)
  ctx_flat = ctx_tokens.reshape(B).astype(jnp.int32)
  neg2d = neg_tokens.reshape(B * NEG // 128, 128).astype(jnp.int32)
  pos, neg = _w2v_call(in_flat, ctx_flat, neg2d,
                       encoder_weight, decoder_weight)
  return pos.reshape(B, 1, 1), neg.reshape(B, 1, NEG)


# speedup vs baseline: 1.0380x; 1.0380x over previous
import jax
import jax.numpy as jnp
from jax import lax
from jax.experimental import pallas as pl
from jax.experimental.pallas import tpu as pltpu
from jax.experimental.pallas import tpu_sc as plsc

B = 16384
DIM = 32
NEG = 20
LANES = 16

NC = 2
NS = 16
NW = NC * NS

B_PER_W = B // NW
CB = 128
NCH = B_PER_W // CB
NEG_CB = CB * NEG
NEG_STREAMS = NEG_CB // 128


def _w2v_body(in_hbm, ctx_hbm, neg_hbm, enc_hbm, dec_hbm,
              pos_out, neg_out,
              widx, cidx, nidx, wrows, crows, nrows, posbuf, negbuf, sem):
  wid = lax.axis_index("s") * NC + lax.axis_index("c")

  def chunk_body(c, _):
    base = pl.multiple_of((wid * NCH + c) * CB, CB)
    pltpu.sync_copy(in_hbm.at[pl.ds(base, CB)], widx)
    pltpu.sync_copy(ctx_hbm.at[pl.ds(base, CB)], cidx)
    pltpu.sync_copy(neg_hbm.at[pl.ds(base * NEG, NEG_CB)], nidx)

    handles = [
        pltpu.async_copy(enc_hbm.at[widx], wrows, sem),
        pltpu.async_copy(dec_hbm.at[cidx], crows, sem),
    ]
    for j in range(NEG_STREAMS):
      handles.append(
          pltpu.async_copy(dec_hbm.at[nidx.at[pl.ds(j * 128, 128)]],
                           nrows.at[pl.ds(j * 128, 128)], sem))
    for h in handles:
      h.wait()

    lane = lax.iota(jnp.int32, LANES)
    _dn = lax.GatherDimensionNumbers(
        offset_dims=(), collapsed_slice_dims=(0,), start_index_map=(0,))
    _pidx = {s: jnp.bitwise_xor(lane, s)[:, None] for s in (8, 4, 2, 1)}
    _pmask = {s: (lane & s) == 0 for s in (8, 4, 2, 1)}

    def _perm(v, s):
      return lax.gather(v, _pidx[s], _dn, (1,),
                        mode=lax.GatherScatterMode.PROMISE_IN_BOUNDS)

    def _combine(a, b, s):
      m = _pmask[s]
      return jnp.where(m, a, _perm(b, s)) + jnp.where(m, _perm(a, s), b)

    def b_body(i, _):
      w0 = wrows[i, pl.ds(0, LANES)]
      w1 = wrows[i, pl.ds(LANES, LANES)]
      c0 = crows[i, pl.ds(0, LANES)]
      c1 = crows[i, pl.ds(LANES, LANES)]
      p = w0 * c0 + w1 * c1
      for s in (8, 4, 2, 1):
        p = p + _perm(p, s)
      posbuf[pl.ds(i, LANES)] = p
      r0 = i * NEG
      h = []
      for k in range(NEG):
        n0 = nrows[r0 + k, pl.ds(0, LANES)]
        n1 = nrows[r0 + k, pl.ds(LANES, LANES)]
        h.append(w0 * n0 + w1 * n1)
      vs = h[:LANES]
      for s in (8, 4, 2, 1):
        half = len(vs) // 2
        vs = [_combine(vs[j], vs[j + half], s) for j in range(half)]
      negbuf[pl.ds(r0, LANES)] = vs[0]
      rs = []
      for k in range(LANES, NEG):
        t = h[k]
        t = t + _perm(t, 8)
        t = t + _perm(t, 4)
        rs.append(t)
      rs = [_combine(rs[j], rs[j + 2], 2) for j in range(2)]
      rs = [_combine(rs[0], rs[1], 1)]
      negbuf[pl.ds(r0 + LANES, LANES)] = rs[0]
      return 0

    lax.fori_loop(0, CB, b_body, 0)

    pltpu.sync_copy(posbuf.at[pl.ds(0, CB)], pos_out.at[pl.ds(base, CB)])
    pltpu.sync_copy(negbuf.at[pl.ds(0, NEG_CB)],
                    neg_out.at[pl.ds(base * NEG, NEG_CB)])
    return 0

  lax.fori_loop(0, NCH, chunk_body, 0)


@jax.jit
def _w2v_call(in_flat, ctx_flat, neg2d, enc, dec):
  mesh = plsc.VectorSubcoreMesh(core_axis_name="c", subcore_axis_name="s")
  kern = pl.kernel(
      _w2v_body,
      out_type=(
          jax.ShapeDtypeStruct((B,), jnp.float32),
          jax.ShapeDtypeStruct((B * NEG,), jnp.float32),
      ),
      mesh=mesh,
      scratch_types=[
          pltpu.VMEM((CB,), jnp.int32),
          pltpu.VMEM((CB,), jnp.int32),
          pltpu.VMEM((NEG_CB,), jnp.int32),
          pltpu.VMEM((CB, DIM), jnp.float32),
          pltpu.VMEM((CB, DIM), jnp.float32),
          pltpu.VMEM((NEG_CB, DIM), jnp.float32),
          pltpu.VMEM((CB + LANES,), jnp.float32),
          pltpu.VMEM((NEG_CB + LANES,), jnp.float32),
          pltpu.SemaphoreType.DMA,
      ],
      compiler_params=pltpu.CompilerParams(use_tc_tiling_on_sc=False),
  )
  return kern(in_flat, ctx_flat, neg2d, enc, dec)


def kernel(input_tokens, ctx_tokens, neg_tokens, encoder_weight, decoder_weight):
  in_flat = input_tokens.reshape(B).astype(jnp.int32)
  ctx_flat = ctx_tokens.reshape(B).astype(jnp.int32)
  neg_flat = neg_tokens.reshape(B * NEG).astype(jnp.int32)
  pos, neg = _w2v_call(in_flat, ctx_flat, neg_flat,
                       encoder_weight, decoder_weight)
  return pos.reshape(B, 1, 1), neg.reshape(B, 1, NEG)
